# baseline (speedup 1.0000x reference)
import functools

import jax
import jax.numpy as jnp
from jax import lax
from jax.experimental import pallas as pl
from jax.experimental.pallas import tpu as pltpu
from jax.experimental.pallas import tpu_sc as plsc

N = 10000
D = 128
E = 320000
NC, NS, CH = 2, 16, 64
P = 4
G = 16
NCHUNK = 320
NG = NCHUNK // G
EPT = NCHUNK * CH
EP = EPT * NS
NACC = 10240
ZPT = NACC // NS
BM = 1000
NB = (2 * N) // BM


def _make_sc_scatter(d, gather=True):
    mesh = plsc.VectorSubcoreMesh(core_axis_name="c", subcore_axis_name="s")
    scratch = (
        [pltpu.VMEM((G, CH), jnp.int32)]
        + [pltpu.VMEM((CH, d), jnp.float32) for _ in range(P if gather else 1)]
        + [pltpu.VMEM_SHARED((NACC, d), jnp.float32)]
        + [pltpu.SemaphoreType.DMA] * (2 * P)
    )
    if gather:
        scratch = [pltpu.VMEM((G, CH), jnp.int32)] + scratch

    def body(*refs):
        if gather:
            (table_hbm, src_hbm, dst_hbm, out_hbm, src_v, dst_v) = refs[:6]
            rows = refs[6:6 + P]
            acc_sh = refs[6 + P]
            sg = refs[7 + P:7 + 2 * P]
            ss = refs[7 + 2 * P:7 + 3 * P]
        else:
            (table_hbm, dst_hbm, out_hbm, dst_v) = refs[:4]
            rows = (refs[4],) * P
            acc_sh = refs[5]
            sg = refs[6:6 + P]
            ss = refs[6 + P:6 + 2 * P]
        c = lax.axis_index("c")
        s = lax.axis_index("s")
        rows0 = rows[0]
        fill_val = jnp.zeros((16,), jnp.float32)

        def fill(i, carry):
            for j in range(d // 16):
                rows0[i, pl.ds(j * 16, 16)] = fill_val
            return carry

        lax.fori_loop(0, CH, fill, 0)
        for z in range(ZPT // CH):
            pltpu.sync_copy(rows0,
                            acc_sh.at[pl.ds(s * ZPT + z * CH, CH)])
        if not gather:
            pltpu.sync_copy(table_hbm, rows0)
        plsc.subcore_barrier()

        def gath(k, b):
            pltpu.async_copy(table_hbm.at[src_v.at[k]], rows[b], sg[b])

        def wait_g(b):
            pltpu.make_async_copy(table_hbm.at[src_v.at[0]],
                                  rows[b], sg[b]).wait()

        def scat(k, b):
            pltpu.async_copy(rows[b], acc_sh.at[dst_v.at[k]], ss[b],
                             add=True)

        def wait_s(b):
            pltpu.make_async_copy(rows[b], acc_sh.at[dst_v.at[0]],
                                  ss[b]).wait()

        def group(g, carry):
            if gather:
                pltpu.sync_copy(src_hbm.at[c, s, pl.ds(g * G, G)], src_v)
            pltpu.sync_copy(dst_hbm.at[c, s, pl.ds(g * G, G)], dst_v)
            for k in range(G):
                b = k % P
                if k >= P:
                    wait_s(b)
                else:
                    @pl.when(g > 0)
                    def _(b=b):
                        wait_s(b)
                if gather:
                    gath(k, b)
                    if k >= P - 1:
                        b2 = (k - (P - 1)) % P
                        wait_g(b2)
                        scat(k - (P - 1), b2)
                else:
                    scat(k, b)
            if gather:
                for k in range(G - P + 1, G):
                    b2 = k % P
                    wait_g(b2)
                    scat(k, b2)
            return carry

        lax.fori_loop(0, NG, group, 0)
        for b in range(P):
            wait_s(b)
        plsc.subcore_barrier()
        pltpu.sync_copy(acc_sh.at[pl.ds(s * ZPT, ZPT)],
                        out_hbm.at[c, pl.ds(s * ZPT, ZPT)])

    return pl.kernel(
        body,
        mesh=mesh,
        out_type=jax.ShapeDtypeStruct((NC, NACC, d), jnp.float32),
        scratch_types=scratch,
    )


def _row_spec(i):
    return (i // (NB // 2), i % (NB // 2), 0)


def _dinv(deg_ref):
    return lax.rsqrt(deg_ref[0, :, 0:1] + 1.0)


def _mm0_body(x_ref, w_ref, deg_ref, o_ref):
    dinv = _dinv(deg_ref)
    h = jnp.dot(x_ref[...], w_ref[...], preferred_element_type=jnp.float32)
    o_ref[...] = h * dinv


def _comb_body(acc_ref, hs_ref, deg_ref, b_ref, w_ref, o_ref):
    dinv = _dinv(deg_ref)
    y = dinv * (acc_ref[0] + hs_ref[...]) + b_ref[...]
    nrm = jnp.maximum(jnp.sqrt(jnp.sum(y * y, axis=1, keepdims=True)), 1e-12)
    y = jnp.maximum(y / nrm, 0.0)
    h = jnp.dot(y, w_ref[...], preferred_element_type=jnp.float32)
    o_ref[...] = h * dinv


def _final_body(acc_ref, hs_ref, deg_ref, b_ref, h_ref, st_ref):
    i = pl.program_id(0)
    dinv = _dinv(deg_ref)
    y = dinv * (acc_ref[0] + hs_ref[...]) + b_ref[...]
    h_ref[0] = y
    s1 = jnp.sum(y, axis=0, keepdims=True)
    s2 = jnp.sum(y * y, axis=0, keepdims=True)
    st = jnp.concatenate([s1, s2], axis=0)[None]

    @pl.when(i % (NB // 2) == 0)
    def _():
        st_ref[...] = st

    @pl.when(i % (NB // 2) != 0)
    def _():
        st_ref[...] += st


def _z_body(h_ref, st_ref, o_ref):
    mean = st_ref[0, 0:1, :] * (1.0 / N)
    ssq = st_ref[0, 1:2, :]
    var = (ssq - N * mean * mean) * (1.0 / (N - 1))
    o_ref[0] = (h_ref[0] - mean) * lax.rsqrt(var)


_BS = pl.BlockSpec


def _tc_mm0(x, w, deg):
    return pl.pallas_call(
        _mm0_body,
        grid=(NB,),
        in_specs=[
            _BS((BM, D), lambda i: (i, 0)),
            _BS((D, D), lambda i: (0, 0)),
            _BS((1, BM, D), _row_spec),
        ],
        out_specs=_BS((BM, D), lambda i: (i, 0)),
        out_shape=jax.ShapeDtypeStruct((2 * N, D), jnp.float32),
    )(x, w, deg)


def _tc_comb(acc, hs, deg, b, w):
    return pl.pallas_call(
        _comb_body,
        grid=(NB,),
        in_specs=[
            _BS((1, BM, D), _row_spec),
            _BS((BM, D), lambda i: (i, 0)),
            _BS((1, BM, D), _row_spec),
            _BS((1, D), lambda i: (0, 0)),
            _BS((D, D), lambda i: (0, 0)),
        ],
        out_specs=_BS((BM, D), lambda i: (i, 0)),
        out_shape=jax.ShapeDtypeStruct((2 * N, D), jnp.float32),
    )(acc, hs, deg, b.reshape(1, D), w)


def _tc_final(acc, hs, deg, b):
    return pl.pallas_call(
        _final_body,
        grid=(NB,),
        in_specs=[
            _BS((1, BM, D), _row_spec),
            _BS((BM, D), lambda i: (i, 0)),
            _BS((1, BM, D), _row_spec),
            _BS((1, D), lambda i: (0, 0)),
        ],
        out_specs=[
            _BS((1, BM, D), _row_spec),
            _BS((1, 2, D), lambda i: (i // (NB // 2), 0, 0)),
        ],
        out_shape=[
            jax.ShapeDtypeStruct((2, N, D), jnp.float32),
            jax.ShapeDtypeStruct((2, 2, D), jnp.float32),
        ],
    )(acc, hs, deg, b.reshape(1, D))


def _tc_z(h, st):
    return pl.pallas_call(
        _z_body,
        grid=(NB,),
        in_specs=[
            _BS((1, BM, D), _row_spec),
            _BS((1, 2, D), lambda i: (i // (NB // 2), 0, 0)),
        ],
        out_specs=_BS((1, BM, D), _row_spec),
        out_shape=jax.ShapeDtypeStruct((2, N, D), jnp.float32),
    )(h, st)


def _prep_edges(ei, src_off):
    pad = EP - E
    fill = jnp.arange(pad, dtype=jnp.int32)
    src = jnp.concatenate([ei[0] + src_off, fill % N + src_off])
    dst = jnp.concatenate([ei[1], N + fill % (NACC - N)])
    return src.reshape(NS, NCHUNK, CH), dst.reshape(NS, NCHUNK, CH)


def kernel(x1, edge_index1, x2, edge_index2, W0, b0, W1, b1, W2, b2):
    src1, dst1 = _prep_edges(edge_index1, 0)
    src2, dst2 = _prep_edges(edge_index2, N)
    src = jnp.stack([src1, src2])
    dst = jnp.stack([dst1, dst2])

    ones_tab = jnp.ones((CH, D), jnp.float32)
    deg = _make_sc_scatter(D, gather=False)(ones_tab, dst)

    x = jnp.concatenate([x1, x2], axis=0)
    msg = _make_sc_scatter(D)

    hs = _tc_mm0(x, W0, deg)
    acc = msg(hs, src, dst)
    hs = _tc_comb(acc, hs, deg, b0, W1)
    acc = msg(hs, src, dst)
    hs = _tc_comb(acc, hs, deg, b1, W2)
    acc = msg(hs, src, dst)
    h, st = _tc_final(acc, hs, deg, b2)
    z = _tc_z(h, st)
    return (z[0], z[1])


# speedup vs baseline: 1.0070x; 1.0070x over previous
import functools

import jax
import jax.numpy as jnp
from jax import lax
from jax.experimental import pallas as pl
from jax.experimental.pallas import tpu as pltpu
from jax.experimental.pallas import tpu_sc as plsc

N = 10000
D = 128
E = 320000
NC, NS, CH = 2, 16, 128
P = 2
G = 16
NCHUNK = 160
NG = NCHUNK // G
EPT = NCHUNK * CH
EP = EPT * NS
NACC = 10240
ZPT = NACC // NS
BM = 1000
NB = (2 * N) // BM


def _make_sc_scatter(d, gather=True):
    mesh = plsc.VectorSubcoreMesh(core_axis_name="c", subcore_axis_name="s")
    scratch = (
        [pltpu.VMEM((G, CH), jnp.int32)]
        + [pltpu.VMEM((CH, d), jnp.float32) for _ in range(P if gather else 1)]
        + [pltpu.VMEM_SHARED((NACC, d), jnp.float32)]
        + [pltpu.SemaphoreType.DMA] * (2 * P)
    )
    if gather:
        scratch = [pltpu.VMEM((G, CH), jnp.int32)] + scratch

    def body(*refs):
        if gather:
            (table_hbm, src_hbm, dst_hbm, out_hbm, src_v, dst_v) = refs[:6]
            rows = refs[6:6 + P]
            acc_sh = refs[6 + P]
            sg = refs[7 + P:7 + 2 * P]
            ss = refs[7 + 2 * P:7 + 3 * P]
        else:
            (table_hbm, dst_hbm, out_hbm, dst_v) = refs[:4]
            rows = (refs[4],) * P
            acc_sh = refs[5]
            sg = refs[6:6 + P]
            ss = refs[6 + P:6 + 2 * P]
        c = lax.axis_index("c")
        s = lax.axis_index("s")
        rows0 = rows[0]
        fill_val = jnp.zeros((16,), jnp.float32)

        def fill(i, carry):
            for j in range(d // 16):
                rows0[i, pl.ds(j * 16, 16)] = fill_val
            return carry

        lax.fori_loop(0, CH, fill, 0)
        for z in range(ZPT // CH):
            pltpu.sync_copy(rows0,
                            acc_sh.at[pl.ds(s * ZPT + z * CH, CH)])
        if not gather:
            pltpu.sync_copy(table_hbm, rows0)
        plsc.subcore_barrier()

        def gath(k, b):
            pltpu.async_copy(table_hbm.at[src_v.at[k]], rows[b], sg[b])

        def wait_g(b):
            pltpu.make_async_copy(table_hbm.at[src_v.at[0]],
                                  rows[b], sg[b]).wait()

        def scat(k, b):
            pltpu.async_copy(rows[b], acc_sh.at[dst_v.at[k]], ss[b],
                             add=True)

        def wait_s(b):
            pltpu.make_async_copy(rows[b], acc_sh.at[dst_v.at[0]],
                                  ss[b]).wait()

        def group(g, carry):
            if gather:
                pltpu.sync_copy(src_hbm.at[c, s, pl.ds(g * G, G)], src_v)
            pltpu.sync_copy(dst_hbm.at[c, s, pl.ds(g * G, G)], dst_v)
            for k in range(G):
                b = k % P
                if k >= P:
                    wait_s(b)
                else:
                    @pl.when(g > 0)
                    def _(b=b):
                        wait_s(b)
                if gather:
                    gath(k, b)
                    if k >= P - 1:
                        b2 = (k - (P - 1)) % P
                        wait_g(b2)
                        scat(k - (P - 1), b2)
                else:
                    scat(k, b)
            if gather:
                for k in range(G - P + 1, G):
                    b2 = k % P
                    wait_g(b2)
                    scat(k, b2)
            return carry

        lax.fori_loop(0, NG, group, 0)
        for b in range(P):
            wait_s(b)
        plsc.subcore_barrier()
        pltpu.sync_copy(acc_sh.at[pl.ds(s * ZPT, ZPT)],
                        out_hbm.at[c, pl.ds(s * ZPT, ZPT)])

    return pl.kernel(
        body,
        mesh=mesh,
        out_type=jax.ShapeDtypeStruct((NC, NACC, d), jnp.float32),
        scratch_types=scratch,
    )


def _row_spec(i):
    return (i // (NB // 2), i % (NB // 2), 0)


def _dinv(deg_ref):
    return lax.rsqrt(deg_ref[0, :, 0:1] + 1.0)


def _mm0_body(x_ref, w_ref, deg_ref, o_ref):
    dinv = _dinv(deg_ref)
    h = jnp.dot(x_ref[...], w_ref[...], preferred_element_type=jnp.float32)
    o_ref[...] = h * dinv


def _comb_body(acc_ref, hs_ref, deg_ref, b_ref, w_ref, o_ref):
    dinv = _dinv(deg_ref)
    y = dinv * (acc_ref[0] + hs_ref[...]) + b_ref[...]
    nrm = jnp.maximum(jnp.sqrt(jnp.sum(y * y, axis=1, keepdims=True)), 1e-12)
    y = jnp.maximum(y / nrm, 0.0)
    h = jnp.dot(y, w_ref[...], preferred_element_type=jnp.float32)
    o_ref[...] = h * dinv


def _final_body(acc_ref, hs_ref, deg_ref, b_ref, h_ref, st_ref):
    i = pl.program_id(0)
    dinv = _dinv(deg_ref)
    y = dinv * (acc_ref[0] + hs_ref[...]) + b_ref[...]
    h_ref[0] = y
    s1 = jnp.sum(y, axis=0, keepdims=True)
    s2 = jnp.sum(y * y, axis=0, keepdims=True)
    st = jnp.concatenate([s1, s2], axis=0)[None]

    @pl.when(i % (NB // 2) == 0)
    def _():
        st_ref[...] = st

    @pl.when(i % (NB // 2) != 0)
    def _():
        st_ref[...] += st


def _z_body(h_ref, st_ref, o_ref):
    mean = st_ref[0, 0:1, :] * (1.0 / N)
    ssq = st_ref[0, 1:2, :]
    var = (ssq - N * mean * mean) * (1.0 / (N - 1))
    o_ref[0] = (h_ref[0] - mean) * lax.rsqrt(var)


_BS = pl.BlockSpec


def _tc_mm0(x, w, deg):
    return pl.pallas_call(
        _mm0_body,
        grid=(NB,),
        in_specs=[
            _BS((BM, D), lambda i: (i, 0)),
            _BS((D, D), lambda i: (0, 0)),
            _BS((1, BM, D), _row_spec),
        ],
        out_specs=_BS((BM, D), lambda i: (i, 0)),
        out_shape=jax.ShapeDtypeStruct((2 * N, D), jnp.float32),
    )(x, w, deg)


def _tc_comb(acc, hs, deg, b, w):
    return pl.pallas_call(
        _comb_body,
        grid=(NB,),
        in_specs=[
            _BS((1, BM, D), _row_spec),
            _BS((BM, D), lambda i: (i, 0)),
            _BS((1, BM, D), _row_spec),
            _BS((1, D), lambda i: (0, 0)),
            _BS((D, D), lambda i: (0, 0)),
        ],
        out_specs=_BS((BM, D), lambda i: (i, 0)),
        out_shape=jax.ShapeDtypeStruct((2 * N, D), jnp.float32),
    )(acc, hs, deg, b.reshape(1, D), w)


def _tc_final(acc, hs, deg, b):
    return pl.pallas_call(
        _final_body,
        grid=(NB,),
        in_specs=[
            _BS((1, BM, D), _row_spec),
            _BS((BM, D), lambda i: (i, 0)),
            _BS((1, BM, D), _row_spec),
            _BS((1, D), lambda i: (0, 0)),
        ],
        out_specs=[
            _BS((1, BM, D), _row_spec),
            _BS((1, 2, D), lambda i: (i // (NB // 2), 0, 0)),
        ],
        out_shape=[
            jax.ShapeDtypeStruct((2, N, D), jnp.float32),
            jax.ShapeDtypeStruct((2, 2, D), jnp.float32),
        ],
    )(acc, hs, deg, b.reshape(1, D))


def _tc_z(h, st):
    return pl.pallas_call(
        _z_body,
        grid=(NB,),
        in_specs=[
            _BS((1, BM, D), _row_spec),
            _BS((1, 2, D), lambda i: (i // (NB // 2), 0, 0)),
        ],
        out_specs=_BS((1, BM, D), _row_spec),
        out_shape=jax.ShapeDtypeStruct((2, N, D), jnp.float32),
    )(h, st)


def _prep_edges(ei, src_off):
    pad = EP - E
    fill = jnp.arange(pad, dtype=jnp.int32)
    src = jnp.concatenate([ei[0] + src_off, fill % N + src_off])
    dst = jnp.concatenate([ei[1], N + fill % (NACC - N)])
    return src.reshape(NS, NCHUNK, CH), dst.reshape(NS, NCHUNK, CH)


def kernel(x1, edge_index1, x2, edge_index2, W0, b0, W1, b1, W2, b2):
    src1, dst1 = _prep_edges(edge_index1, 0)
    src2, dst2 = _prep_edges(edge_index2, N)
    src = jnp.stack([src1, src2])
    dst = jnp.stack([dst1, dst2])

    ones_tab = jnp.ones((CH, D), jnp.float32)
    deg = _make_sc_scatter(D, gather=False)(ones_tab, dst)

    x = jnp.concatenate([x1, x2], axis=0)
    msg = _make_sc_scatter(D)

    hs = _tc_mm0(x, W0, deg)
    acc = msg(hs, src, dst)
    hs = _tc_comb(acc, hs, deg, b0, W1)
    acc = msg(hs, src, dst)
    hs = _tc_comb(acc, hs, deg, b1, W2)
    acc = msg(hs, src, dst)
    h, st = _tc_final(acc, hs, deg, b2)
    z = _tc_z(h, st)
    return (z[0], z[1])
